# baseline (speedup 1.0000x reference)
import functools

import jax
import jax.numpy as jnp
from jax import lax
from jax.experimental import pallas as pl
from jax.experimental.pallas import tpu as pltpu
from jax.experimental.pallas import tpu_sc as plsc

_B = 32
_N = 32768
_R = _N // 2
_ROWS = 256
_LANES = 128


def _sort_body(xyzT_ref, dir_ref, idx_ref):
    d0 = dir_ref[0, 0, 0]
    d1 = dir_ref[0, 0, 1]
    d2 = dir_ref[0, 0, 2]
    norm = jnp.sqrt((d0 * d0 + d1 * d1) + d2 * d2) + jnp.float32(1e-12)
    n0 = d0 / norm
    n1 = d1 / norm
    n2 = d2 / norm

    X = xyzT_ref[0].reshape(3, _ROWS, _LANES)
    xb0 = X[0].astype(jnp.bfloat16).astype(jnp.float32)
    xb1 = X[1].astype(jnp.bfloat16).astype(jnp.float32)
    xb2 = X[2].astype(jnp.bfloat16).astype(jnp.float32)
    nb0 = jnp.float32(jnp.bfloat16(n0))
    nb1 = jnp.float32(jnp.bfloat16(n1))
    nb2 = jnp.float32(jnp.bfloat16(n2))
    s = xb0 * nb0 + (xb1 * nb1 + xb2 * nb2)

    b = s.view(jnp.int32)
    m = b >> 31
    K = b ^ (m & jnp.int32(0x7FFFFFFF))

    flat = (lax.broadcasted_iota(jnp.int32, (_ROWS, _LANES), 0) * _LANES
            + lax.broadcasted_iota(jnp.int32, (_ROWS, _LANES), 1))
    I = flat

    n = _ROWS * _LANES
    k = 2
    while k <= n:
        j = k // 2
        while j >= 1:
            if j < _LANES:
                axis, amt = 1, j
            else:
                axis, amt = 0, j // _LANES
            Kp = jnp.roll(K, -amt, axis)
            Km = jnp.roll(K, amt, axis)
            Ip = jnp.roll(I, -amt, axis)
            Im = jnp.roll(I, amt, axis)
            is_lower = (flat & j) == 0
            Kpart = jnp.where(is_lower, Kp, Km)
            Ipart = jnp.where(is_lower, Ip, Im)
            self_first = (K > Kpart) | ((K == Kpart) & (I < Ipart))
            dir_desc = (flat & k) == 0
            keep = self_first == (dir_desc == is_lower)
            K = jnp.where(keep, K, Kpart)
            I = jnp.where(keep, I, Ipart)
            j //= 2
        k *= 2

    idx_ref[0] = I[: _R // _LANES]


def _topk_indices(xyzT, direction):
    return pl.pallas_call(
        _sort_body,
        grid=(_B,),
        in_specs=[
            pl.BlockSpec((1, 3, _N), lambda b: (b, 0, 0)),
            pl.BlockSpec((1, 1, 3), lambda b: (b, 0, 0), memory_space=pltpu.SMEM),
        ],
        out_specs=pl.BlockSpec((1, _R // _LANES, _LANES), lambda b: (b, 0, 0)),
        out_shape=jax.ShapeDtypeStruct((_B, _R // _LANES, _LANES), jnp.int32),
    )(xyzT, direction.reshape(_B, 1, 3))


_CHUNK = 2048
_NCHUNK = _R // _CHUNK


def _gather_body(xyzT_hbm, idx_hbm, out_hbm, xv, yv, zv, idxv, stage):
    w = lax.axis_index("s") * 2 + lax.axis_index("c")
    pltpu.sync_copy(xyzT_hbm.at[w * 3], xv)
    pltpu.sync_copy(xyzT_hbm.at[w * 3 + 1], yv)
    pltpu.sync_copy(xyzT_hbm.at[w * 3 + 2], zv)
    pltpu.sync_copy(idx_hbm.at[w], idxv)
    iota3 = lax.iota(jnp.int32, 16) * 3

    for chunk in range(_NCHUNK):
        def grp(g, carry):
            p = idxv[pl.ds(chunk * _CHUNK + g * 16, 16)]
            base = g * 48 + iota3
            plsc.store_scatter(stage, [base], plsc.load_gather(xv, [p]))
            plsc.store_scatter(stage, [base + 1], plsc.load_gather(yv, [p]))
            plsc.store_scatter(stage, [base + 2], plsc.load_gather(zv, [p]))
            return carry

        lax.fori_loop(0, _CHUNK // 16, grp, 0)
        pltpu.sync_copy(
            stage, out_hbm.at[w].at[pl.ds(chunk * _CHUNK * 3, _CHUNK * 3)])


_gather = functools.partial(
    pl.kernel,
    mesh=plsc.VectorSubcoreMesh(core_axis_name="c", subcore_axis_name="s"),
    out_type=jax.ShapeDtypeStruct((_B, _R * 3), jnp.float32),
    scratch_types=[
        pltpu.VMEM((_N,), jnp.float32),
        pltpu.VMEM((_N,), jnp.float32),
        pltpu.VMEM((_N,), jnp.float32),
        pltpu.VMEM((_R,), jnp.int32),
        pltpu.VMEM((_CHUNK * 3,), jnp.float32),
    ],
    compiler_params=pltpu.CompilerParams(needs_layout_passes=False),
)(_gather_body)


def kernel(xyz, direction):
    xyzT = jnp.swapaxes(xyz, 1, 2)
    idx = _topk_indices(xyzT, direction)
    out = _gather(xyzT.reshape(_B * 3, _N), idx.reshape(_B, _R))
    return out.reshape(_B, _R, 3)


# speedup vs baseline: 1.1168x; 1.1168x over previous
import functools

import jax
import jax.numpy as jnp
from jax import lax
from jax.experimental import pallas as pl
from jax.experimental.pallas import tpu as pltpu
from jax.experimental.pallas import tpu_sc as plsc

_B = 32
_N = 32768
_R = _N // 2
_ROWS = 256
_LANES = 128


def _sort_body(xyzT_ref, dir_ref, idx_ref):
    d0 = dir_ref[0, 0, 0]
    d1 = dir_ref[0, 0, 1]
    d2 = dir_ref[0, 0, 2]
    norm = jnp.sqrt((d0 * d0 + d1 * d1) + d2 * d2) + jnp.float32(1e-12)
    n0 = d0 / norm
    n1 = d1 / norm
    n2 = d2 / norm

    X = xyzT_ref[0].reshape(3, _ROWS, _LANES)
    xb0 = X[0].astype(jnp.bfloat16).astype(jnp.float32)
    xb1 = X[1].astype(jnp.bfloat16).astype(jnp.float32)
    xb2 = X[2].astype(jnp.bfloat16).astype(jnp.float32)
    nb0 = jnp.float32(jnp.bfloat16(n0))
    nb1 = jnp.float32(jnp.bfloat16(n1))
    nb2 = jnp.float32(jnp.bfloat16(n2))
    s = xb0 * nb0 + (xb1 * nb1 + xb2 * nb2)

    b = s.view(jnp.int32)
    m = b >> 31
    K = b ^ (m & jnp.int32(0x7FFFFFFF))

    flat = (lax.broadcasted_iota(jnp.int32, (_ROWS, _LANES), 0) * _LANES
            + lax.broadcasted_iota(jnp.int32, (_ROWS, _LANES), 1))
    I = flat

    n = _ROWS * _LANES

    def step(K, I, flat, j):
        if j < _LANES:
            axis, amt = 1, j
        else:
            axis, amt = 0, j // _LANES
        Kp = jnp.roll(K, -amt, axis)
        Km = jnp.roll(K, amt, axis)
        Ip = jnp.roll(I, -amt, axis)
        Im = jnp.roll(I, amt, axis)
        is_lower = (flat & j) == 0
        Kpart = jnp.where(is_lower, Kp, Km)
        Ipart = jnp.where(is_lower, Ip, Im)
        self_first = (K > Kpart) | ((K == Kpart) & (I < Ipart))
        keep = self_first == is_lower
        return jnp.where(keep, K, Kpart), jnp.where(keep, I, Ipart)

    k = 2
    while k < n:
        Dk = jnp.where((flat & k) == 0, jnp.int32(0), jnp.int32(-1))
        K = K ^ Dk
        I = I ^ Dk
        j = k // 2
        while j >= 1:
            K, I = step(K, I, flat, j)
            j //= 2
        K = K ^ Dk
        I = I ^ Dk
        k *= 2

    H = _ROWS // 2
    KA, KB, IA, IB = K[:H], K[H:], I[:H], I[H:]
    a_first = (KA > KB) | ((KA == KB) & (IA < IB))
    K = jnp.where(a_first, KA, KB)
    I = jnp.where(a_first, IA, IB)
    flat = flat[:H]
    j = n // 4
    while j >= 1:
        K, I = step(K, I, flat, j)
        j //= 2

    idx_ref[0] = I


def _topk_indices(xyzT, direction):
    return pl.pallas_call(
        _sort_body,
        grid=(_B,),
        in_specs=[
            pl.BlockSpec((1, 3, _N), lambda b: (b, 0, 0)),
            pl.BlockSpec((1, 1, 3), lambda b: (b, 0, 0), memory_space=pltpu.SMEM),
        ],
        out_specs=pl.BlockSpec((1, _R // _LANES, _LANES), lambda b: (b, 0, 0)),
        out_shape=jax.ShapeDtypeStruct((_B, _R // _LANES, _LANES), jnp.int32),
    )(xyzT, direction.reshape(_B, 1, 3))


_CHUNK = 2048
_NCHUNK = _R // _CHUNK


def _gather_body(xyzT_hbm, idx_hbm, out_hbm, xv, yv, zv, idxv, stage):
    w = lax.axis_index("s") * 2 + lax.axis_index("c")
    pltpu.sync_copy(xyzT_hbm.at[w * 3], xv)
    pltpu.sync_copy(xyzT_hbm.at[w * 3 + 1], yv)
    pltpu.sync_copy(xyzT_hbm.at[w * 3 + 2], zv)
    pltpu.sync_copy(idx_hbm.at[w], idxv)
    iota3 = lax.iota(jnp.int32, 16) * 3

    for chunk in range(_NCHUNK):
        def grp(g, carry):
            p = idxv[pl.ds(chunk * _CHUNK + g * 16, 16)]
            base = g * 48 + iota3
            plsc.store_scatter(stage, [base], plsc.load_gather(xv, [p]))
            plsc.store_scatter(stage, [base + 1], plsc.load_gather(yv, [p]))
            plsc.store_scatter(stage, [base + 2], plsc.load_gather(zv, [p]))
            return carry

        lax.fori_loop(0, _CHUNK // 16, grp, 0)
        pltpu.sync_copy(
            stage, out_hbm.at[w].at[pl.ds(chunk * _CHUNK * 3, _CHUNK * 3)])


_gather = functools.partial(
    pl.kernel,
    mesh=plsc.VectorSubcoreMesh(core_axis_name="c", subcore_axis_name="s"),
    out_type=jax.ShapeDtypeStruct((_B, _R * 3), jnp.float32),
    scratch_types=[
        pltpu.VMEM((_N,), jnp.float32),
        pltpu.VMEM((_N,), jnp.float32),
        pltpu.VMEM((_N,), jnp.float32),
        pltpu.VMEM((_R,), jnp.int32),
        pltpu.VMEM((_CHUNK * 3,), jnp.float32),
    ],
    compiler_params=pltpu.CompilerParams(needs_layout_passes=False),
)(_gather_body)


def kernel(xyz, direction):
    xyzT = jnp.swapaxes(xyz, 1, 2)
    idx = _topk_indices(xyzT, direction)
    out = _gather(xyzT.reshape(_B * 3, _N), idx.reshape(_B, _R))
    return out.reshape(_B, _R, 3)


# speedup vs baseline: 1.2488x; 1.1182x over previous
import functools

import jax
import jax.numpy as jnp
from jax import lax
from jax.experimental import pallas as pl
from jax.experimental.pallas import tpu as pltpu
from jax.experimental.pallas import tpu_sc as plsc

_B = 32
_N = 32768
_R = _N // 2
_ROWS = 256
_LANES = 128


def _sort_body(xyzT_ref, dir_ref, idx_ref):
    Kt = []
    for t in range(2):
        d0 = dir_ref[t, 0, 0]
        d1 = dir_ref[t, 0, 1]
        d2 = dir_ref[t, 0, 2]
        norm = jnp.sqrt((d0 * d0 + d1 * d1) + d2 * d2) + jnp.float32(1e-12)
        n0 = d0 / norm
        n1 = d1 / norm
        n2 = d2 / norm

        X = xyzT_ref[t].reshape(3, _ROWS, _LANES)
        xb0 = X[0].astype(jnp.bfloat16).astype(jnp.float32)
        xb1 = X[1].astype(jnp.bfloat16).astype(jnp.float32)
        xb2 = X[2].astype(jnp.bfloat16).astype(jnp.float32)
        nb0 = jnp.float32(jnp.bfloat16(n0))
        nb1 = jnp.float32(jnp.bfloat16(n1))
        nb2 = jnp.float32(jnp.bfloat16(n2))
        s = xb0 * nb0 + (xb1 * nb1 + xb2 * nb2)

        b = s.view(jnp.int32)
        m = b >> 31
        Kt.append(b ^ (m & jnp.int32(0x7FFFFFFF)))

    K = jnp.stack(Kt)
    flat = (lax.broadcasted_iota(jnp.int32, (_ROWS, _LANES), 0) * _LANES
            + lax.broadcasted_iota(jnp.int32, (_ROWS, _LANES), 1))
    I = jnp.broadcast_to(flat, (2, _ROWS, _LANES))

    n = _ROWS * _LANES

    def step(K, I, flat, j):
        if j < _LANES:
            axis, amt = 2, j
        else:
            axis, amt = 1, j // _LANES
        Kp = jnp.roll(K, -amt, axis)
        Km = jnp.roll(K, amt, axis)
        Ip = jnp.roll(I, -amt, axis)
        Im = jnp.roll(I, amt, axis)
        is_lower = (flat & j) == 0
        Kpart = jnp.where(is_lower, Kp, Km)
        Ipart = jnp.where(is_lower, Ip, Im)
        self_first = (K > Kpart) | ((K == Kpart) & (I < Ipart))
        keep = self_first == is_lower
        return jnp.where(keep, K, Kpart), jnp.where(keep, I, Ipart)

    k = 2
    while k < n:
        Dk = jnp.where((flat & k) == 0, jnp.int32(0), jnp.int32(-1))
        K = K ^ Dk
        I = I ^ Dk
        j = k // 2
        while j >= 1:
            K, I = step(K, I, flat, j)
            j //= 2
        K = K ^ Dk
        I = I ^ Dk
        k *= 2

    H = _ROWS // 2
    KA, KB, IA, IB = K[:, :H], K[:, H:], I[:, :H], I[:, H:]
    a_first = (KA > KB) | ((KA == KB) & (IA < IB))
    K = jnp.where(a_first, KA, KB)
    I = jnp.where(a_first, IA, IB)
    flat = flat[:H]
    j = n // 4
    while j >= 1:
        K, I = step(K, I, flat, j)
        j //= 2

    idx_ref[...] = I


def _topk_indices(xyzT, direction):
    return pl.pallas_call(
        _sort_body,
        grid=(_B // 2,),
        in_specs=[
            pl.BlockSpec((2, 3, _N), lambda b: (b, 0, 0)),
            pl.BlockSpec((2, 1, 3), lambda b: (b, 0, 0), memory_space=pltpu.SMEM),
        ],
        out_specs=pl.BlockSpec((2, _R // _LANES, _LANES), lambda b: (b, 0, 0)),
        out_shape=jax.ShapeDtypeStruct((_B, _R // _LANES, _LANES), jnp.int32),
    )(xyzT, direction.reshape(_B, 1, 3))


_CHUNK = 2048
_NCHUNK = _R // _CHUNK


def _gather_body(xyzT_hbm, idx_hbm, out_hbm, xv, yv, zv, idxv, stage):
    w = lax.axis_index("s") * 2 + lax.axis_index("c")
    pltpu.sync_copy(xyzT_hbm.at[w * 3], xv)
    pltpu.sync_copy(xyzT_hbm.at[w * 3 + 1], yv)
    pltpu.sync_copy(xyzT_hbm.at[w * 3 + 2], zv)
    pltpu.sync_copy(idx_hbm.at[w], idxv)
    iota3 = lax.iota(jnp.int32, 16) * 3

    for chunk in range(_NCHUNK):
        def grp(g, carry):
            p = idxv[pl.ds(chunk * _CHUNK + g * 16, 16)]
            base = g * 48 + iota3
            plsc.store_scatter(stage, [base], plsc.load_gather(xv, [p]))
            plsc.store_scatter(stage, [base + 1], plsc.load_gather(yv, [p]))
            plsc.store_scatter(stage, [base + 2], plsc.load_gather(zv, [p]))
            return carry

        lax.fori_loop(0, _CHUNK // 16, grp, 0)
        pltpu.sync_copy(
            stage, out_hbm.at[w].at[pl.ds(chunk * _CHUNK * 3, _CHUNK * 3)])


_gather = functools.partial(
    pl.kernel,
    mesh=plsc.VectorSubcoreMesh(core_axis_name="c", subcore_axis_name="s"),
    out_type=jax.ShapeDtypeStruct((_B, _R * 3), jnp.float32),
    scratch_types=[
        pltpu.VMEM((_N,), jnp.float32),
        pltpu.VMEM((_N,), jnp.float32),
        pltpu.VMEM((_N,), jnp.float32),
        pltpu.VMEM((_R,), jnp.int32),
        pltpu.VMEM((_CHUNK * 3,), jnp.float32),
    ],
    compiler_params=pltpu.CompilerParams(needs_layout_passes=False),
)(_gather_body)


def kernel(xyz, direction):
    xyzT = jnp.swapaxes(xyz, 1, 2)
    idx = _topk_indices(xyzT, direction)
    out = _gather(xyzT.reshape(_B * 3, _N), idx.reshape(_B, _R))
    return out.reshape(_B, _R, 3)
